# Initial kernel scaffold; baseline (speedup 1.0000x reference)
import jax
import jax.numpy as jnp
from jax.experimental import pallas as pl


def kernel(edge_index, emb, W0, a_src0, a_dst0, b0, W1, a_src1, a_dst1, b1, W2, a_src2, a_dst2, b2):
    raise NotImplementedError("write your pallas kernel here")


# speedup vs baseline: 1.2435x; 1.2435x over previous
import functools

import jax
import jax.numpy as jnp
from jax import lax
from jax.experimental import pallas as pl
from jax.experimental.pallas import tpu as pltpu

N_NODES = 50000
D = 64
_R = 512


def _dense_body(x_ref, w_ref, as_ref, ad_ref, h_ref, al_ref, ar_ref, c_ref):
    i = pl.program_id(0)
    h = jnp.dot(x_ref[...], w_ref[...], preferred_element_type=jnp.float32)
    h_ref[...] = h
    al = jnp.sum(h * as_ref[...], axis=1, keepdims=True)
    ar = jnp.sum(h * ad_ref[...], axis=1, keepdims=True)
    al_ref[...] = al
    ar_ref[...] = ar
    blk_c = jnp.full((1, 1), jnp.max(al) + jnp.max(ar), dtype=jnp.float32)

    @pl.when(i == 0)
    def _():
        c_ref[...] = jnp.zeros_like(c_ref)

    c_ref[...] = jnp.maximum(c_ref[...], blk_c)


def _dense_layer(x_pad, W, a_s, a_d):
    n_pad = x_pad.shape[0]
    grid = n_pad // _R
    h, al, ar, c = pl.pallas_call(
        _dense_body,
        grid=(grid,),
        in_specs=[
            pl.BlockSpec((_R, D), lambda i: (i, 0)),
            pl.BlockSpec((D, D), lambda i: (0, 0)),
            pl.BlockSpec((1, D), lambda i: (0, 0)),
            pl.BlockSpec((1, D), lambda i: (0, 0)),
        ],
        out_specs=[
            pl.BlockSpec((_R, D), lambda i: (i, 0)),
            pl.BlockSpec((_R, 1), lambda i: (i, 0)),
            pl.BlockSpec((_R, 1), lambda i: (i, 0)),
            pl.BlockSpec((1, 1), lambda i: (0, 0)),
        ],
        out_shape=[
            jax.ShapeDtypeStruct((n_pad, D), jnp.float32),
            jax.ShapeDtypeStruct((n_pad, 1), jnp.float32),
            jax.ShapeDtypeStruct((n_pad, 1), jnp.float32),
            jax.ShapeDtypeStruct((1, 1), jnp.float32),
        ],
    )(x_pad, W, a_s.reshape(1, D), a_d.reshape(1, D))
    return h, al[:, 0], ar[:, 0], jnp.maximum(c[0, 0], 0.0)


def kernel(edge_index, emb, W0, a_src0, a_dst0, b0, W1, a_src1, a_dst1, b1,
           W2, a_src2, a_dst2, b2):
    n = emb.shape[0]
    n_pad = ((n + _R - 1) // _R) * _R
    loops = jnp.arange(n, dtype=edge_index.dtype)
    ei = jnp.concatenate([edge_index, jnp.stack([loops, loops], axis=0)], axis=1)
    src = ei[0].astype(jnp.int32)
    dst = ei[1].astype(jnp.int32)

    x = jnp.pad(emb, ((0, n_pad - n), (0, 0)))
    acc = emb

    for (W, a_s, a_d, b) in ((W0, a_src0, a_dst0, b0), (W1, a_src1, a_dst1, b1),
                             (W2, a_src2, a_dst2, b2)):
        h, al, ar, C = _dense_layer(x, W, a_s, a_d)
        e = jax.nn.leaky_relu(al[src] + ar[dst], negative_slope=0.2)
        ex = jnp.exp(e - C)
        den = jax.ops.segment_sum(ex, dst, num_segments=n)
        alpha = ex / (den[dst] + 1e-16)
        g = jax.ops.segment_sum(h[src] * alpha[:, None], dst, num_segments=n)
        xl = jax.nn.elu(g + b)
        acc = acc + xl
        x = jnp.pad(xl, ((0, n_pad - n), (0, 0)))

    return acc / 4.0


# speedup vs baseline: 28.5710x; 22.9762x over previous
import functools

import jax
import jax.numpy as jnp
from jax import lax
from jax.experimental import pallas as pl
from jax.experimental.pallas import tpu as pltpu
from jax.experimental.pallas import tpu_sc as plsc

N_NODES = 50000
D = 64
DH = 16
NP = 51200
TS = NP // 16
E_TOT = 850000
E_PAD = 851968
EROWS = E_PAD // 128
_R = 512

B1 = 2048
NCH1 = E_PAD // (32 * B1)
B2 = 1024
NCH2 = E_PAD // (16 * B2)

_mesh = plsc.VectorSubcoreMesh(core_axis_name="c", subcore_axis_name="s")


def _split_h_store(h_ref, h):
    for q in range(4):
        h_ref[q] = h[:, q * DH:(q + 1) * DH]


def _attn_stats(h, as_ref, ad_ref, al_ref, ar_ref, c_ref, i):
    al = jnp.sum(h * as_ref[...], axis=1, keepdims=True)
    ar = jnp.sum(h * ad_ref[...], axis=1, keepdims=True)
    al_ref[...] = al
    ar_ref[...] = ar
    blk_c = jnp.full((1, 128), jnp.max(al) + jnp.max(ar), dtype=jnp.float32)

    @pl.when(i == 0)
    def _():
        c_ref[...] = jnp.zeros_like(c_ref)

    c_ref[...] = jnp.maximum(c_ref[...], blk_c)


def _tc_first_body(x_ref, w_ref, as_ref, ad_ref, h_ref, al_ref, ar_ref, c_ref):
    i = pl.program_id(0)
    h = jnp.dot(x_ref[...], w_ref[...], preferred_element_type=jnp.float32)
    _split_h_store(h_ref, h)
    _attn_stats(h, as_ref, ad_ref, al_ref, ar_ref, c_ref, i)


def _x_from_g(g0_ref, g1_ref, g2_ref, g3_ref, d0_ref, d1_ref, b_ref, i):
    g = jnp.concatenate([g0_ref[0], g1_ref[0], g2_ref[0], g3_ref[0]],
                        axis=1)
    den = d0_ref[0] + d1_ref[0]
    x = g / (den + 1e-16) + b_ref[...]
    x = jnp.where(x > 0.0, x, jnp.exp(x) - 1.0)
    rows = i * _R + lax.broadcasted_iota(jnp.int32, (_R, 1), 0)
    return jnp.where(rows < N_NODES, x, 0.0)


def _tc_mid_body(g0_ref, g1_ref, g2_ref, g3_ref, d0_ref, d1_ref, b_ref,
                 acc_ref, w_ref, as_ref, ad_ref, h_ref, al_ref, ar_ref,
                 c_ref, accout_ref):
    i = pl.program_id(0)
    x = _x_from_g(g0_ref, g1_ref, g2_ref, g3_ref, d0_ref, d1_ref, b_ref, i)
    accout_ref[...] = acc_ref[...] + x
    h = jnp.dot(x, w_ref[...], preferred_element_type=jnp.float32)
    _split_h_store(h_ref, h)
    _attn_stats(h, as_ref, ad_ref, al_ref, ar_ref, c_ref, i)


def _tc_last_body(ga0_ref, ga1_ref, gb0_ref, gb1_ref, d0_ref, d1_ref, b_ref,
                  acc_ref, out_ref):
    i = pl.program_id(0)
    x = _x_from_g(ga0_ref, ga1_ref, gb0_ref, gb1_ref, d0_ref, d1_ref, b_ref, i)
    out_ref[...] = (acc_ref[...] + x) * 0.25


def _dense_outs():
    return [
        jax.ShapeDtypeStruct((4, NP, DH), jnp.float32),
        jax.ShapeDtypeStruct((NP, 1), jnp.float32),
        jax.ShapeDtypeStruct((NP, 1), jnp.float32),
        jax.ShapeDtypeStruct((1, 128), jnp.float32),
    ]


def _dense_out_specs():
    return [
        pl.BlockSpec((4, _R, DH), lambda i: (0, i, 0)),
        pl.BlockSpec((_R, 1), lambda i: (i, 0)),
        pl.BlockSpec((_R, 1), lambda i: (i, 0)),
        pl.BlockSpec((1, 128), lambda i: (0, 0)),
    ]


def _g_den_specs():
    return [
        pl.BlockSpec((1, _R, DH), lambda i: (0, i, 0)),
        pl.BlockSpec((1, _R, DH), lambda i: (1, i, 0)),
        pl.BlockSpec((1, _R, DH), lambda i: (0, i, 0)),
        pl.BlockSpec((1, _R, DH), lambda i: (1, i, 0)),
        pl.BlockSpec((1, _R, 1), lambda i: (0, i, 0)),
        pl.BlockSpec((1, _R, 1), lambda i: (1, i, 0)),
        pl.BlockSpec((1, D), lambda i: (0, 0)),
    ]


def _tc_first(x_pad, W, a_s, a_d):
    grid = NP // _R
    return pl.pallas_call(
        _tc_first_body,
        grid=(grid,),
        in_specs=[
            pl.BlockSpec((_R, D), lambda i: (i, 0)),
            pl.BlockSpec((D, D), lambda i: (0, 0)),
            pl.BlockSpec((1, D), lambda i: (0, 0)),
            pl.BlockSpec((1, D), lambda i: (0, 0)),
        ],
        out_specs=_dense_out_specs(),
        out_shape=_dense_outs(),
    )(x_pad, W, a_s.reshape(1, D), a_d.reshape(1, D))


def _tc_mid(ga3, gb3, den3, b, acc, W, a_s, a_d):
    grid = NP // _R
    return pl.pallas_call(
        _tc_mid_body,
        grid=(grid,),
        in_specs=_g_den_specs() + [
            pl.BlockSpec((_R, D), lambda i: (i, 0)),
            pl.BlockSpec((D, D), lambda i: (0, 0)),
            pl.BlockSpec((1, D), lambda i: (0, 0)),
            pl.BlockSpec((1, D), lambda i: (0, 0)),
        ],
        out_specs=_dense_out_specs() + [pl.BlockSpec((_R, D), lambda i: (i, 0))],
        out_shape=_dense_outs() + [jax.ShapeDtypeStruct((NP, D), jnp.float32)],
    )(ga3, ga3, gb3, gb3, den3, den3, b.reshape(1, D), acc, W,
      a_s.reshape(1, D), a_d.reshape(1, D))


def _tc_last(ga3, gb3, den3, b, acc):
    grid = NP // _R
    return pl.pallas_call(
        _tc_last_body,
        grid=(grid,),
        in_specs=_g_den_specs() + [pl.BlockSpec((_R, D), lambda i: (i, 0))],
        out_specs=pl.BlockSpec((_R, D), lambda i: (i, 0)),
        out_shape=jax.ShapeDtypeStruct((NP, D), jnp.float32),
    )(ga3, ga3, gb3, gb3, den3, den3, b.reshape(1, D), acc)


def _sc_pass1_body(src_hbm, dst_hbm, al_hbm, ar_hbm, c_hbm, zden_hbm,
                   ex_hbm, den_hbm,
                   idx_s, idx_d, va, vb, ex_v, c_v, den_acc, sem):
    c = lax.axis_index("c")
    s = lax.axis_index("s")
    wid = s * 2 + c
    pltpu.sync_copy(zden_hbm, den_acc.at[pl.ds(s * TS, TS)])
    pltpu.sync_copy(c_hbm, c_v)
    plsc.subcore_barrier()
    cvec = c_v[0, pl.ds(0, 16)]

    def chunk(i, carry):
        base = (wid * NCH1 + i) * B1
        rbase = (wid * NCH1 + i) * (B1 // 128)
        pltpu.sync_copy(src_hbm.at[pl.ds(rbase, B1 // 128)], idx_s)
        pltpu.sync_copy(dst_hbm.at[pl.ds(rbase, B1 // 128)], idx_d)

        def fire(j, carry2):
            pltpu.make_async_copy(al_hbm.at[idx_s.at[j]],
                                  va.at[pl.ds(j * 128, 128)], sem).start()
            pltpu.make_async_copy(ar_hbm.at[idx_d.at[j]],
                                  vb.at[pl.ds(j * 128, 128)], sem).start()
            return carry2

        lax.fori_loop(0, B1 // 128, fire, 0)

        def drain(j, carry2):
            pltpu.make_async_copy(al_hbm.at[idx_s.at[j]],
                                  va.at[pl.ds(j * 128, 128)], sem).wait()
            pltpu.make_async_copy(ar_hbm.at[idx_d.at[j]],
                                  vb.at[pl.ds(j * 128, 128)], sem).wait()
            return carry2

        lax.fori_loop(0, B1 // 128, drain, 0)

        def cmp(k, carry2):
            z = va[pl.ds(k * 16, 16)] + vb[pl.ds(k * 16, 16)]
            e = jnp.where(z > 0.0, z, 0.2 * z)
            ex_v[pl.ds(k * 16, 16)] = jnp.exp(e - cvec)
            return carry2

        lax.fori_loop(0, B1 // 16, cmp, 0)
        pltpu.sync_copy(ex_v, ex_hbm.at[pl.ds(base, B1)])

        def scat(j, carry2):
            pltpu.sync_copy(ex_v.at[pl.ds(j * 128, 128)],
                            den_acc.at[idx_d.at[j]], add=True)
            return carry2

        lax.fori_loop(0, B1 // 128, scat, 0)
        return carry

    lax.fori_loop(0, NCH1, chunk, 0)
    plsc.subcore_barrier()
    pltpu.sync_copy(den_acc.at[pl.ds(s * TS, TS)],
                    den_hbm.at[pl.ds(c * NP + s * TS, TS)])


def _make_pass2(qbase):
    def body(srcadj_hbm, dst_hbm, ex_hbm, h_hbm, zacc_hbm,
             g_hbm,
             idx_s, idx_d, ex_v, rows, acc, sem):
        c = lax.axis_index("c")
        s = lax.axis_index("s")
        pltpu.sync_copy(zacc_hbm, acc.at[pl.ds(s * TS, TS)])
        plsc.subcore_barrier()

        def chunk(i, carry):
            base = (s * NCH2 + i) * B2
            rbase = (s * NCH2 + i) * (B2 // 128)
            pltpu.sync_copy(
                srcadj_hbm.at[pl.ds((qbase + c) * EROWS + rbase, B2 // 128)],
                idx_s)
            pltpu.sync_copy(dst_hbm.at[pl.ds(rbase, B2 // 128)], idx_d)
            pltpu.sync_copy(ex_hbm.at[pl.ds(base, B2)], ex_v)

            def fire(j, carry2):
                pltpu.make_async_copy(h_hbm.at[idx_s.at[j]],
                                      rows.at[pl.ds(j * 128, 128)],
                                      sem).start()
                return carry2

            lax.fori_loop(0, B2 // 128, fire, 0)

            def drain(j, carry2):
                pltpu.make_async_copy(h_hbm.at[idx_s.at[j]],
                                      rows.at[pl.ds(j * 128, 128)],
                                      sem).wait()
                return carry2

            lax.fori_loop(0, B2 // 128, drain, 0)

            def scale(r0, carry2):
                exv = ex_v[pl.ds(r0 * 16, 16)]
                for t in range(16):
                    av = exv[t]
                    row = r0 * 16 + t
                    rows[row, pl.ds(0, DH)] = rows[row, pl.ds(0, DH)] * av
                return carry2

            lax.fori_loop(0, B2 // 16, scale, 0)

            def scat(j, carry2):
                pltpu.sync_copy(rows.at[pl.ds(j * 128, 128)],
                                acc.at[idx_d.at[j]], add=True)
                return carry2

            lax.fori_loop(0, B2 // 128, scat, 0)
            return carry

        lax.fori_loop(0, NCH2, chunk, 0)
        plsc.subcore_barrier()
        pltpu.sync_copy(acc.at[pl.ds(s * TS, TS)],
                        g_hbm.at[pl.ds(c * NP + s * TS, TS)])

    return pl.kernel(
        body,
        out_type=jax.ShapeDtypeStruct((2 * NP, DH), jnp.float32),
        mesh=_mesh,
        scratch_types=[
            pltpu.VMEM((B2 // 128, 128), jnp.int32),
            pltpu.VMEM((B2 // 128, 128), jnp.int32),
            pltpu.VMEM((B2,), jnp.float32),
            pltpu.VMEM((B2, DH), jnp.float32),
            pltpu.VMEM_SHARED((NP, DH), jnp.float32),
            pltpu.SemaphoreType.DMA,
        ],
        compiler_params=pltpu.CompilerParams(use_tc_tiling_on_sc=False),
    )


_sc_pass1 = pl.kernel(
    _sc_pass1_body,
    out_type=[
        jax.ShapeDtypeStruct((E_PAD,), jnp.float32),
        jax.ShapeDtypeStruct((2 * NP,), jnp.float32),
    ],
    mesh=_mesh,
    scratch_types=[
        pltpu.VMEM((B1 // 128, 128), jnp.int32),
        pltpu.VMEM((B1 // 128, 128), jnp.int32),
        pltpu.VMEM((B1,), jnp.float32),
        pltpu.VMEM((B1,), jnp.float32),
        pltpu.VMEM((B1,), jnp.float32),
        pltpu.VMEM((1, 128), jnp.float32),
        pltpu.VMEM_SHARED((NP,), jnp.float32),
        pltpu.SemaphoreType.DMA,
    ],
)

_sc_pass2_a = _make_pass2(0)
_sc_pass2_b = _make_pass2(2)


def kernel(edge_index, emb, W0, a_src0, a_dst0, b0, W1, a_src1, a_dst1, b1,
           W2, a_src2, a_dst2, b2):
    n = emb.shape[0]
    loops = jnp.arange(n, dtype=edge_index.dtype)
    src = jnp.concatenate([edge_index[0], loops]).astype(jnp.int32)
    dst = jnp.concatenate([edge_index[1], loops]).astype(jnp.int32)
    src = jnp.pad(src, (0, E_PAD - E_TOT), constant_values=n)
    dst = jnp.pad(dst, (0, E_PAD - E_TOT), constant_values=n)
    src2d = src.reshape(EROWS, 128)
    dst2d = dst.reshape(EROWS, 128)
    srcadj = jnp.concatenate(
        [src2d + q * NP for q in range(4)], axis=0)

    zden = jnp.zeros((TS,), jnp.float32)
    zacc = jnp.zeros((TS, DH), jnp.float32)

    x = jnp.pad(emb, ((0, NP - n), (0, 0)))
    acc = x

    h4, al, ar, cb = _tc_first(x, W0, a_src0, a_dst0)
    layers = ((W1, a_src1, a_dst1, b0), (W2, a_src2, a_dst2, b1))
    for (Wn, a_sn, a_dn, b_prev) in layers:
        ex, den = _sc_pass1(src2d, dst2d, al.reshape(NP), ar.reshape(NP),
                            cb, zden)
        hq = h4.reshape(4 * NP, DH)
        ga = _sc_pass2_a(srcadj, dst2d, ex, hq, zacc)
        gb = _sc_pass2_b(srcadj, dst2d, ex, hq, zacc)
        den3 = den.reshape(2, NP, 1)
        h4, al, ar, cb, acc = _tc_mid(ga.reshape(2, NP, DH),
                                      gb.reshape(2, NP, DH), den3, b_prev,
                                      acc, Wn, a_sn, a_dn)

    ex, den = _sc_pass1(src2d, dst2d, al.reshape(NP), ar.reshape(NP), cb, zden)
    hq = h4.reshape(4 * NP, DH)
    ga = _sc_pass2_a(srcadj, dst2d, ex, hq, zacc)
    gb = _sc_pass2_b(srcadj, dst2d, ex, hq, zacc)
    out = _tc_last(ga.reshape(2, NP, DH), gb.reshape(2, NP, DH),
                   den.reshape(2, NP, 1), b2, acc)
    return out[:n]


# speedup vs baseline: 37.0653x; 1.2973x over previous
import jax
import jax.numpy as jnp
from jax import lax
from jax.experimental import pallas as pl
from jax.experimental.pallas import tpu as pltpu
from jax.experimental.pallas import tpu_sc as plsc

N_NODES = 50000
D = 64
DH = 16
NP = 51200
TS = NP // 16
E_TOT = 850000
E_PAD = 851968
EROWS = E_PAD // 128
_R = 512

B1 = 2048
NCH1 = E_PAD // (32 * B1)
B2 = 1024
EPT = E_PAD // 16
NCH2 = EPT // B2
HCH = NCH2 // 2
MROWS = HCH * (B2 // 128)

_mesh = plsc.VectorSubcoreMesh(core_axis_name="c", subcore_axis_name="s")


def _split_h_store(h_ref, h):
    for q in range(4):
        h_ref[q] = h[:, q * DH:(q + 1) * DH]


def _attn_stats(h, as_ref, ad_ref, al_ref, ar_ref, c_ref, i):
    al = jnp.sum(h * as_ref[...], axis=1, keepdims=True)
    ar = jnp.sum(h * ad_ref[...], axis=1, keepdims=True)
    al_ref[...] = al
    ar_ref[...] = ar
    blk_c = jnp.full((1, 128), jnp.max(al) + jnp.max(ar), dtype=jnp.float32)

    @pl.when(i == 0)
    def _():
        c_ref[...] = jnp.zeros_like(c_ref)

    c_ref[...] = jnp.maximum(c_ref[...], blk_c)


def _tc_first_body(x_ref, w_ref, as_ref, ad_ref, h_ref, al_ref, ar_ref, c_ref):
    i = pl.program_id(0)
    h = jnp.dot(x_ref[...], w_ref[...], preferred_element_type=jnp.float32)
    _split_h_store(h_ref, h)
    _attn_stats(h, as_ref, ad_ref, al_ref, ar_ref, c_ref, i)


def _x_from_g(g0_ref, g1_ref, g2_ref, g3_ref, d0_ref, d1_ref, b_ref, i):
    g = jnp.concatenate([g0_ref[0], g1_ref[0], g2_ref[0], g3_ref[0]],
                        axis=1)
    den = d0_ref[0] + d1_ref[0]
    x = g / (den + 1e-16) + b_ref[...]
    x = jnp.where(x > 0.0, x, jnp.exp(x) - 1.0)
    rows = i * _R + lax.broadcasted_iota(jnp.int32, (_R, 1), 0)
    return jnp.where(rows < N_NODES, x, 0.0)


def _tc_mid_body(g0_ref, g1_ref, g2_ref, g3_ref, d0_ref, d1_ref, b_ref,
                 acc_ref, w_ref, as_ref, ad_ref, h_ref, al_ref, ar_ref,
                 c_ref, accout_ref):
    i = pl.program_id(0)
    x = _x_from_g(g0_ref, g1_ref, g2_ref, g3_ref, d0_ref, d1_ref, b_ref, i)
    accout_ref[...] = acc_ref[...] + x
    h = jnp.dot(x, w_ref[...], preferred_element_type=jnp.float32)
    _split_h_store(h_ref, h)
    _attn_stats(h, as_ref, ad_ref, al_ref, ar_ref, c_ref, i)


def _tc_last_body(ga0_ref, ga1_ref, gb0_ref, gb1_ref, d0_ref, d1_ref, b_ref,
                  acc_ref, out_ref):
    i = pl.program_id(0)
    x = _x_from_g(ga0_ref, ga1_ref, gb0_ref, gb1_ref, d0_ref, d1_ref, b_ref, i)
    out_ref[...] = (acc_ref[...] + x) * 0.25


def _dense_outs():
    return [
        jax.ShapeDtypeStruct((4, NP, DH), jnp.float32),
        jax.ShapeDtypeStruct((NP, 1), jnp.float32),
        jax.ShapeDtypeStruct((NP, 1), jnp.float32),
        jax.ShapeDtypeStruct((1, 128), jnp.float32),
    ]


def _dense_out_specs():
    return [
        pl.BlockSpec((4, _R, DH), lambda i: (0, i, 0)),
        pl.BlockSpec((_R, 1), lambda i: (i, 0)),
        pl.BlockSpec((_R, 1), lambda i: (i, 0)),
        pl.BlockSpec((1, 128), lambda i: (0, 0)),
    ]


def _g_den_specs():
    return [
        pl.BlockSpec((1, _R, DH), lambda i: (0, i, 0)),
        pl.BlockSpec((1, _R, DH), lambda i: (1, i, 0)),
        pl.BlockSpec((1, _R, DH), lambda i: (0, i, 0)),
        pl.BlockSpec((1, _R, DH), lambda i: (1, i, 0)),
        pl.BlockSpec((1, _R, 1), lambda i: (0, i, 0)),
        pl.BlockSpec((1, _R, 1), lambda i: (1, i, 0)),
        pl.BlockSpec((1, D), lambda i: (0, 0)),
    ]


def _tc_first(x_pad, W, a_s, a_d):
    grid = NP // _R
    return pl.pallas_call(
        _tc_first_body,
        grid=(grid,),
        in_specs=[
            pl.BlockSpec((_R, D), lambda i: (i, 0)),
            pl.BlockSpec((D, D), lambda i: (0, 0)),
            pl.BlockSpec((1, D), lambda i: (0, 0)),
            pl.BlockSpec((1, D), lambda i: (0, 0)),
        ],
        out_specs=_dense_out_specs(),
        out_shape=_dense_outs(),
    )(x_pad, W, a_s.reshape(1, D), a_d.reshape(1, D))


def _tc_mid(ga3, gb3, den3, b, acc, W, a_s, a_d):
    grid = NP // _R
    return pl.pallas_call(
        _tc_mid_body,
        grid=(grid,),
        in_specs=_g_den_specs() + [
            pl.BlockSpec((_R, D), lambda i: (i, 0)),
            pl.BlockSpec((D, D), lambda i: (0, 0)),
            pl.BlockSpec((1, D), lambda i: (0, 0)),
            pl.BlockSpec((1, D), lambda i: (0, 0)),
        ],
        out_specs=_dense_out_specs() + [pl.BlockSpec((_R, D), lambda i: (i, 0))],
        out_shape=_dense_outs() + [jax.ShapeDtypeStruct((NP, D), jnp.float32)],
    )(ga3, ga3, gb3, gb3, den3, den3, b.reshape(1, D), acc, W,
      a_s.reshape(1, D), a_d.reshape(1, D))


def _tc_last(ga3, gb3, den3, b, acc):
    grid = NP // _R
    return pl.pallas_call(
        _tc_last_body,
        grid=(grid,),
        in_specs=_g_den_specs() + [pl.BlockSpec((_R, D), lambda i: (i, 0))],
        out_specs=pl.BlockSpec((_R, D), lambda i: (i, 0)),
        out_shape=jax.ShapeDtypeStruct((NP, D), jnp.float32),
    )(ga3, ga3, gb3, gb3, den3, den3, b.reshape(1, D), acc)


def _gather_128(i, rows_per_chunk, meta, hbm, dst_v, sem, start):
    for j in range(rows_per_chunk):
        cp = pltpu.make_async_copy(hbm.at[meta.at[i * rows_per_chunk + j]],
                                   dst_v.at[pl.ds(j * 128, 128)], sem)
        if start:
            cp.start()
        else:
            cp.wait()


def _scatter_128(i, rows_per_chunk, meta, src_v, spm, sem, start):
    if not start:
        return
    for j in range(rows_per_chunk):
        pltpu.sync_copy(src_v.at[pl.ds(j * 128, 128)],
                        spm.at[meta.at[i * rows_per_chunk + j]], add=True)


def _scatter_vec(i, rows_per_chunk, meta, src2d, spm, sem, start):
    if not start:
        return
    for j in range(rows_per_chunk):
        pltpu.sync_copy(src2d.at[j],
                        spm.at[meta.at[i * rows_per_chunk + j]], add=True)


def _scale_rows(rows, exv):
    def scale(r0, carry):
        ex16 = exv[r0 // 8, pl.ds((r0 % 8) * 16, 16)]
        for t in range(16):
            row = r0 * 16 + t
            rows[row, pl.ds(0, DH)] = rows[row, pl.ds(0, DH)] * ex16[t]
        return carry

    lax.fori_loop(0, B2 // 16, scale, 0)


def _sc_pass1_body(src_hbm, dst_hbm, al_hbm, ar_hbm, c_hbm, zden_hbm,
                   ex_hbm, den_hbm,
                   meta_s, meta_d, c_v, va, vb, exv, den_acc, sem_g):
    c = lax.axis_index("c")
    s = lax.axis_index("s")
    wid = s * 2 + c
    R1 = B1 // 128
    pltpu.sync_copy(zden_hbm, den_acc.at[pl.ds(s * TS, TS)])
    pltpu.sync_copy(c_hbm, c_v)
    mrow = wid * (NCH1 * R1)
    plsc.subcore_barrier()
    cvec = c_v[0, pl.ds(0, 16)]

    def fire(i, b):
        pltpu.sync_copy(src_hbm.at[pl.ds(mrow + i * R1, R1)], meta_s.at[b])
        pltpu.sync_copy(dst_hbm.at[pl.ds(mrow + i * R1, R1)], meta_d.at[b])
        _gather_128(0, R1, meta_s.at[b], al_hbm, va.at[b], sem_g.at[b], True)
        _gather_128(0, R1, meta_d.at[b], ar_hbm, vb.at[b], sem_g.at[b], True)

    def process(i, b):
        _gather_128(0, R1, meta_s.at[b], al_hbm, va.at[b], sem_g.at[b], False)
        _gather_128(0, R1, meta_d.at[b], ar_hbm, vb.at[b], sem_g.at[b], False)

        def cmp(k, carry):
            z = va[b, pl.ds(k * 16, 16)] + vb[b, pl.ds(k * 16, 16)]
            e = jnp.where(z > 0.0, z, 0.2 * z)
            exv[b, k // 8, pl.ds((k % 8) * 16, 16)] = jnp.exp(e - cvec)
            return carry

        lax.fori_loop(0, B1 // 16, cmp, 0)
        pltpu.sync_copy(exv.at[b], ex_hbm.at[pl.ds(mrow + i * R1, R1)])
        _scatter_vec(0, R1, meta_d.at[b], exv.at[b], den_acc, None, True)

    fire(0, 0)
    fire(1, 1)
    for i in range(NCH1):
        process(i, i % 2)
        if i + 2 < NCH1:
            fire(i + 2, i % 2)

    plsc.subcore_barrier()
    pltpu.sync_copy(den_acc.at[pl.ds(s * TS, TS)],
                    den_hbm.at[pl.ds(c * NP + s * TS, TS)])


_sc_pass1 = pl.kernel(
    _sc_pass1_body,
    out_type=[
        jax.ShapeDtypeStruct((EROWS, 128), jnp.float32),
        jax.ShapeDtypeStruct((2 * NP,), jnp.float32),
    ],
    mesh=_mesh,
    scratch_types=[
        pltpu.VMEM((2, B1 // 128, 128), jnp.int32),
        pltpu.VMEM((2, B1 // 128, 128), jnp.int32),
        pltpu.VMEM((1, 128), jnp.float32),
        pltpu.VMEM((2, B1), jnp.float32),
        pltpu.VMEM((2, B1), jnp.float32),
        pltpu.VMEM((2, B1 // 128, 128), jnp.float32),
        pltpu.VMEM_SHARED((NP,), jnp.float32),
        pltpu.SemaphoreType.DMA((2,)),
    ],
    compiler_params=pltpu.CompilerParams(use_tc_tiling_on_sc=False),
)


def _make_pass2(qbase):
    def body(srcadj_hbm, dst_hbm, ex_hbm, h_hbm, zacc_hbm, dep_hbm,
             g_hbm,
             meta_s, meta_d, exv, rows, acc, sem_g):
        c = lax.axis_index("c")
        s = lax.axis_index("s")
        R2 = B2 // 128
        pltpu.sync_copy(zacc_hbm, acc.at[pl.ds(s * TS, TS)])
        plsc.subcore_barrier()
        mrow0 = s * NCH2 * R2

        def fire(i, b):
            pltpu.sync_copy(
                srcadj_hbm.at[pl.ds((qbase + c) * EROWS + mrow0 + i * R2, R2)],
                meta_s.at[b])
            pltpu.sync_copy(dst_hbm.at[pl.ds(mrow0 + i * R2, R2)],
                            meta_d.at[b])
            _gather_128(0, R2, meta_s.at[b], h_hbm, rows.at[b],
                        sem_g.at[b], True)
            pltpu.async_copy(ex_hbm.at[pl.ds(mrow0 + i * R2, R2)],
                             exv.at[b], sem_g.at[b])

        def process(i, b):
            _gather_128(0, R2, meta_s.at[b], h_hbm, rows.at[b],
                        sem_g.at[b], False)
            pltpu.make_async_copy(ex_hbm.at[pl.ds(mrow0 + i * R2, R2)],
                                  exv.at[b], sem_g.at[b]).wait()
            _scale_rows(rows.at[b], exv.at[b])
            _scatter_128(0, R2, meta_d.at[b], rows.at[b], acc, None, True)

        fire(0, 0)
        fire(1, 1)

        def step(m, carry):
            i = 2 * m
            process(i, 0)

            @pl.when(i + 2 < NCH2)
            def _():
                fire(i + 2, 0)

            process(i + 1, 1)

            @pl.when(i + 3 < NCH2)
            def _():
                fire(i + 3, 1)

            return carry

        lax.fori_loop(0, NCH2 // 2, step, 0)

        plsc.subcore_barrier()
        pltpu.sync_copy(acc.at[pl.ds(s * TS, TS)],
                        g_hbm.at[pl.ds(c * NP + s * TS, TS)])

    return pl.kernel(
        body,
        out_type=jax.ShapeDtypeStruct((2 * NP, DH), jnp.float32),
        mesh=_mesh,
        scratch_types=[
            pltpu.VMEM((2, B2 // 128, 128), jnp.int32),
            pltpu.VMEM((2, B2 // 128, 128), jnp.int32),
            pltpu.VMEM((2, B2 // 128, 128), jnp.float32),
            pltpu.VMEM((2, B2, DH), jnp.float32),
            pltpu.VMEM_SHARED((NP, DH), jnp.float32),
            pltpu.SemaphoreType.DMA((2,)),
        ],
        compiler_params=pltpu.CompilerParams(use_tc_tiling_on_sc=False),
    )


_sc_pass2_a = _make_pass2(0)
_sc_pass2_b = _make_pass2(2)


def kernel(edge_index, emb, W0, a_src0, a_dst0, b0, W1, a_src1, a_dst1, b1,
           W2, a_src2, a_dst2, b2):
    n = emb.shape[0]
    loops = jnp.arange(n, dtype=edge_index.dtype)
    src = jnp.concatenate([edge_index[0], loops]).astype(jnp.int32)
    dst = jnp.concatenate([edge_index[1], loops]).astype(jnp.int32)
    src = jnp.pad(src, (0, E_PAD - E_TOT), constant_values=n)
    dst = jnp.pad(dst, (0, E_PAD - E_TOT), constant_values=n)
    src2d = src.reshape(EROWS, 128)
    dst2d = dst.reshape(EROWS, 128)
    srcadj = jnp.concatenate(
        [src2d + q * NP for q in range(4)], axis=0)

    zden = jnp.zeros((TS,), jnp.float32)
    zacc = jnp.zeros((TS, DH), jnp.float32)

    x = jnp.pad(emb, ((0, NP - n), (0, 0)))
    acc = x

    h4, al, ar, cb = _tc_first(x, W0, a_src0, a_dst0)
    layers = ((W1, a_src1, a_dst1, b0), (W2, a_src2, a_dst2, b1))
    for (Wn, a_sn, a_dn, b_prev) in layers:
        ex, den = _sc_pass1(src2d, dst2d, al.reshape(NP), ar.reshape(NP),
                            cb, zden)
        hq = h4.reshape(4 * NP, DH)
        ga = _sc_pass2_a(srcadj, dst2d, ex, hq, zacc, zacc)
        gb = _sc_pass2_b(srcadj, dst2d, ex, hq, zacc, ga)
        den3 = den.reshape(2, NP, 1)
        h4, al, ar, cb, acc = _tc_mid(ga.reshape(2, NP, DH),
                                      gb.reshape(2, NP, DH), den3, b_prev,
                                      acc, Wn, a_sn, a_dn)

    ex, den = _sc_pass1(src2d, dst2d, al.reshape(NP), ar.reshape(NP), cb, zden)
    hq = h4.reshape(4 * NP, DH)
    ga = _sc_pass2_a(srcadj, dst2d, ex, hq, zacc, zacc)
    gb = _sc_pass2_b(srcadj, dst2d, ex, hq, zacc, ga)
    out = _tc_last(ga.reshape(2, NP, DH), gb.reshape(2, NP, DH),
                   den.reshape(2, NP, 1), b2, acc)
    return out[:n]
